# baseline (speedup 1.0000x reference)
import functools

import jax
import jax.numpy as jnp
from jax import lax
from jax.experimental import pallas as pl
from jax.experimental.pallas import tpu as pltpu
from jax.experimental.pallas import tpu_sc as plsc

NC, NS, L = 2, 16, 16
EPS2 = 1e-16


def _rsqrt(x):
    i = plsc.bitcast(x, jnp.int32)
    y = plsc.bitcast(jnp.int32(0x5F3759DF) - (i >> 1), jnp.float32)
    for _ in range(3):
        y = y * (1.5 - 0.5 * x * y * y)
    return y


def kernel(pred_embs, ground_truth_embs, table, noise, num_sampled, margin):
    P, D = pred_embs.shape
    S = noise.shape[0]
    NW = NC * NS
    B = L
    G = P // (NW * B)
    RPG = B * S
    CHUNKS = []
    off = 0
    while off < RPG:
        n = min(128, RPG - off)
        CHUNKS.append((off, n))
        off += n

    noise_flat = noise.T.reshape(-1)
    margin_vec = jnp.full((L,), margin, dtype=jnp.float32)

    W = D // 2

    def to_words(x):
        n = x.shape[0]
        return lax.bitcast_convert_type(
            x.astype(jnp.bfloat16).reshape(n, W, 2), jnp.int32)

    pred_w = to_words(pred_embs)
    gt_w = to_words(ground_truth_embs)
    table_w = to_words(table)

    mesh = plsc.VectorSubcoreMesh(
        core_axis_name="c", subcore_axis_name="s",
        num_cores=NC, num_subcores=NS)

    @functools.partial(
        pl.kernel,
        out_type=jax.ShapeDtypeStruct((NW * L,), jnp.float32),
        mesh=mesh,
        compiler_params=pltpu.CompilerParams(needs_layout_passes=False, use_tc_tiling_on_sc=False),
        scratch_types=[
            pltpu.VMEM((G * RPG,), jnp.int32),
            pltpu.VMEM((RPG, W), jnp.int32),
            pltpu.VMEM((RPG, W), jnp.int32),
            pltpu.VMEM((B, W), jnp.int32),
            pltpu.VMEM((B, W), jnp.int32),
            pltpu.VMEM((B, W), jnp.int32),
            pltpu.VMEM((B, W), jnp.int32),
            pltpu.VMEM((L,), jnp.float32),
            pltpu.VMEM((L,), jnp.float32),
            pltpu.SemaphoreType.DMA,
            pltpu.SemaphoreType.DMA,
        ],
    )
    def sc_body(pred_hbm, gt_hbm, table_hbm, noise_hbm, margin_hbm, out_hbm,
                idx_v, rows0, rows1, pred0, pred1, gt0, gt1, margin_v,
                out_v, sem0, sem1):
        wid = lax.axis_index("s") * NC + lax.axis_index("c")
        rows_b = [rows0, rows1]
        pred_b = [pred0, pred1]
        gt_b = [gt0, gt1]
        sem_b = [sem0, sem1]

        pltpu.sync_copy(noise_hbm.at[pl.ds(wid * (G * RPG), G * RPG)], idx_v)
        pltpu.sync_copy(margin_hbm, margin_v)
        margin_val = margin_v[...]

        iota = lax.iota(jnp.int32, L)
        row_of_lane = iota * S

        def start_group(g, b):
            base_p = wid * (G * B) + g * B
            for off, n in CHUNKS:
                pltpu.async_copy(
                    table_hbm.at[idx_v.at[pl.ds(g * RPG + off, n)]],
                    rows_b[b].at[pl.ds(off, n), :],
                    sem_b[b])
            pltpu.async_copy(pred_hbm.at[pl.ds(base_p, B), :],
                             pred_b[b], sem_b[b])
            pltpu.async_copy(gt_hbm.at[pl.ds(base_p, B), :],
                             gt_b[b], sem_b[b])

        def wait_group(b):
            pltpu.make_async_copy(
                table_hbm.at[pl.ds(0, RPG), :], rows_b[b], sem_b[b]).wait()
            pltpu.make_async_copy(
                pred_hbm.at[pl.ds(0, B), :], pred_b[b], sem_b[b]).wait()
            pltpu.make_async_copy(
                gt_hbm.at[pl.ds(0, B), :], gt_b[b], sem_b[b]).wait()

        zeros = jnp.zeros((L,), jnp.float32)
        bzeros = jnp.zeros((2 * L,), jnp.bfloat16)

        def pairsum(x):
            a, bb = plsc.unpack(x, format=plsc.PackFormat.INTERLEAVED)
            return a + bb

        def compute(b, acc):
            rows_v, pred_v, gt_v = rows_b[b], pred_b[b], gt_b[b]

            CH = 5
            U = 2

            def make_body(c):
                srows = [row_of_lane + (c * CH + j) for j in range(CH)]

                def body(i, carry):
                    st = list(carry)
                    for u in range(U):
                        col = (iota + (i * U + u)) & (W - 1)
                        pv = plsc.bitcast(
                            plsc.load_gather(pred_v, [iota, col]),
                            jnp.bfloat16)
                        if c == 0:
                            gv = plsc.bitcast(
                                plsc.load_gather(gt_v, [iota, col]),
                                jnp.bfloat16)
                            st[-3] = st[-3] + pv * pv
                            st[-2] = st[-2] + gv * gv
                            st[-1] = st[-1] + pv * gv
                        for j in range(CH):
                            bv = plsc.bitcast(
                                plsc.load_gather(rows_v, [srows[j], col]),
                                jnp.bfloat16)
                            st[2 * j] = st[2 * j] + pv * bv
                            st[2 * j + 1] = st[2 * j + 1] + bv * bv
                    return tuple(st)

                return body

            st0 = lax.fori_loop(0, W // U, make_body(0),
                                (bzeros,) * (2 * CH + 3))
            na2 = pairsum(st0[-3])
            ng2 = pairsum(st0[-2])
            dpg = pairsum(st0[-1])
            cos_t = dpg * _rsqrt(jnp.maximum(na2 * ng2, EPS2))

            cos_n = zeros
            for j in range(CH):
                dot, nb2 = pairsum(st0[2 * j]), pairsum(st0[2 * j + 1])
                cos_n = cos_n + dot * _rsqrt(jnp.maximum(na2 * nb2, EPS2))
            for c in range(1, S // CH):
                st = lax.fori_loop(0, W // U, make_body(c),
                                   (bzeros,) * (2 * CH))
                for j in range(CH):
                    dot, nb2 = pairsum(st[2 * j]), pairsum(st[2 * j + 1])
                    cos_n = cos_n + dot * _rsqrt(jnp.maximum(na2 * nb2, EPS2))

            return acc + jnp.maximum(cos_n - cos_t + margin_val, 0.0)

        start_group(0, 0)

        def gbody(i, acc):
            g = 2 * i
            start_group(g + 1, 1)
            wait_group(0)
            acc = compute(0, acc)
            start_group(jnp.minimum(g + 2, G - 1), 0)
            wait_group(1)
            acc = compute(1, acc)
            return acc

        acc = lax.fori_loop(0, G // 2, gbody, zeros)
        wait_group(0)

        out_v[...] = acc
        pltpu.sync_copy(out_v, out_hbm.at[pl.ds(wid * L, L)])

    partials = sc_body(pred_w, gt_w, table_w, noise_flat, margin_vec)
    return jnp.sum(partials)


# speedup vs baseline: 5.1168x; 5.1168x over previous
import functools

import jax
import jax.numpy as jnp
from jax import lax
from jax.experimental import pallas as pl
from jax.experimental.pallas import tpu as pltpu
from jax.experimental.pallas import tpu_sc as plsc

NC, NS, L = 2, 16, 16
EPS2 = 1e-16


def _rsqrt(x):
    i = plsc.bitcast(x, jnp.int32)
    y = plsc.bitcast(jnp.int32(0x5F3759DF) - (i >> 1), jnp.float32)
    for _ in range(3):
        y = y * (1.5 - 0.5 * x * y * y)
    return y


def kernel(pred_embs, ground_truth_embs, table, noise, num_sampled, margin):
    P, D = pred_embs.shape
    S = noise.shape[0]
    NW = NC * NS
    B = L
    G = P // (NW * B)
    RPG = B * S
    CHUNKS = []
    off = 0
    while off < RPG:
        n = min(128, RPG - off)
        CHUNKS.append((off, n))
        off += n

    noise_flat = noise.T.reshape(-1)
    margin_vec = jnp.full((L,), margin, dtype=jnp.float32)

    mesh = plsc.VectorSubcoreMesh(
        core_axis_name="c", subcore_axis_name="s",
        num_cores=NC, num_subcores=NS)

    @functools.partial(
        pl.kernel,
        out_type=jax.ShapeDtypeStruct((NW * L,), jnp.float32),
        mesh=mesh,
        compiler_params=pltpu.CompilerParams(needs_layout_passes=False),
        scratch_types=[
            pltpu.VMEM((G * RPG,), jnp.int32),
            pltpu.VMEM((RPG, D), jnp.float32),
            pltpu.VMEM((RPG, D), jnp.float32),
            pltpu.VMEM((B, D), jnp.float32),
            pltpu.VMEM((B, D), jnp.float32),
            pltpu.VMEM((B, D), jnp.float32),
            pltpu.VMEM((B, D), jnp.float32),
            pltpu.VMEM((L,), jnp.float32),
            pltpu.VMEM((L,), jnp.float32),
            pltpu.SemaphoreType.DMA,
            pltpu.SemaphoreType.DMA,
        ],
    )
    def sc_body(pred_hbm, gt_hbm, table_hbm, noise_hbm, margin_hbm, out_hbm,
                idx_v, rows0, rows1, pred0, pred1, gt0, gt1, margin_v,
                out_v, sem0, sem1):
        wid = lax.axis_index("s") * NC + lax.axis_index("c")
        rows_b = [rows0, rows1]
        pred_b = [pred0, pred1]
        gt_b = [gt0, gt1]
        sem_b = [sem0, sem1]

        pltpu.sync_copy(noise_hbm.at[pl.ds(wid * (G * RPG), G * RPG)], idx_v)
        pltpu.sync_copy(margin_hbm, margin_v)
        margin_val = margin_v[...]

        iota = lax.iota(jnp.int32, L)
        row_of_lane = iota * S

        def start_group(g, b):
            base_p = wid * (G * B) + g * B
            for off, n in CHUNKS:
                pltpu.async_copy(
                    table_hbm.at[idx_v.at[pl.ds(g * RPG + off, n)]],
                    rows_b[b].at[pl.ds(off, n), :],
                    sem_b[b])
            pltpu.async_copy(pred_hbm.at[pl.ds(base_p, B), :],
                             pred_b[b], sem_b[b])
            pltpu.async_copy(gt_hbm.at[pl.ds(base_p, B), :],
                             gt_b[b], sem_b[b])

        def wait_group(b):
            pltpu.make_async_copy(
                table_hbm.at[pl.ds(0, RPG), :], rows_b[b], sem_b[b]).wait()
            pltpu.make_async_copy(
                pred_hbm.at[pl.ds(0, B), :], pred_b[b], sem_b[b]).wait()
            pltpu.make_async_copy(
                gt_hbm.at[pl.ds(0, B), :], gt_b[b], sem_b[b]).wait()

        zeros = jnp.zeros((L,), jnp.float32)
        bzeros = jnp.zeros((2 * L,), jnp.bfloat16)

        def pairsum(x):
            a, bb = plsc.unpack(x, format=plsc.PackFormat.INTERLEAVED)
            return a + bb

        def compute(b, acc):
            rows_v, pred_v, gt_v = rows_b[b], pred_b[b], gt_b[b]

            CH = 5
            U = 2
            W = D // 2

            def pk(ref, r0, c0, c1):
                lo = plsc.load_gather(ref, [r0, c0])
                hi = plsc.load_gather(ref, [r0, c1])
                return plsc.pack(lo, hi, format=plsc.PackFormat.INTERLEAVED)

            def make_body(c):
                srows = [row_of_lane + (c * CH + j) for j in range(CH)]

                def body(i, carry):
                    st = list(carry)
                    for u in range(U):
                        col0 = (iota + 2 * (i * U + u)) & (D - 1)
                        col1 = (col0 + 1) & (D - 1)
                        pv = pk(pred_v, iota, col0, col1)
                        if c == 0:
                            gv = pk(gt_v, iota, col0, col1)
                            st[-3] = st[-3] + pv * pv
                            st[-2] = st[-2] + gv * gv
                            st[-1] = st[-1] + pv * gv
                        for j in range(CH):
                            bv = pk(rows_v, srows[j], col0, col1)
                            st[2 * j] = st[2 * j] + pv * bv
                            st[2 * j + 1] = st[2 * j + 1] + bv * bv
                    return tuple(st)

                return body

            st0 = lax.fori_loop(0, W // U, make_body(0),
                                (bzeros,) * (2 * CH + 3))
            na2 = pairsum(st0[-3])
            ng2 = pairsum(st0[-2])
            dpg = pairsum(st0[-1])
            cos_t = dpg * _rsqrt(jnp.maximum(na2 * ng2, EPS2))

            cos_n = zeros
            for j in range(CH):
                dot, nb2 = pairsum(st0[2 * j]), pairsum(st0[2 * j + 1])
                cos_n = cos_n + dot * _rsqrt(jnp.maximum(na2 * nb2, EPS2))
            for c in range(1, S // CH):
                st = lax.fori_loop(0, W // U, make_body(c),
                                   (bzeros,) * (2 * CH))
                for j in range(CH):
                    dot, nb2 = pairsum(st[2 * j]), pairsum(st[2 * j + 1])
                    cos_n = cos_n + dot * _rsqrt(jnp.maximum(na2 * nb2, EPS2))

            return acc + jnp.maximum(cos_n - cos_t + margin_val, 0.0)

        start_group(0, 0)

        def gbody(i, acc):
            g = 2 * i
            start_group(g + 1, 1)
            wait_group(0)
            acc = compute(0, acc)
            start_group(jnp.minimum(g + 2, G - 1), 0)
            wait_group(1)
            acc = compute(1, acc)
            return acc

        acc = lax.fori_loop(0, G // 2, gbody, zeros)
        wait_group(0)

        out_v[...] = acc
        pltpu.sync_copy(out_v, out_hbm.at[pl.ds(wid * L, L)])

    partials = sc_body(pred_embs, ground_truth_embs, table, noise_flat,
                       margin_vec)
    return jnp.sum(partials)
